# baseline (speedup 1.0000x reference)
import functools
import math

import jax
import jax.numpy as jnp
from jax import lax
from jax.experimental import pallas as pl
from jax.experimental.pallas import tpu as pltpu
from jax.experimental.pallas import tpu_sc as plsc

_D = 128
_U = 128
_G = 8
_GM = 8
_NC, _NS = 2, 16
_NW = _NC * _NS
_CH = 128
_NEG = -(2 ** 31)


def _gather_body(n_ch, table_hbm, idx_hbm, out_hbm, idx_v, rows_v, sem):
    wid = lax.axis_index("s") * _NC + lax.axis_index("c")
    pltpu.sync_copy(idx_hbm.at[pl.ds(wid * n_ch, n_ch)], idx_v)
    for j in range(n_ch):
        pltpu.async_copy(table_hbm.at[idx_v.at[j]], rows_v, sem).wait()
        pltpu.sync_copy(rows_v, out_hbm.at[pl.ds((wid * n_ch + j) * _CH, _CH)])


def _row_gather(table, idx2d):
    g = idx2d.shape[0]
    n_ch = g // _NW
    mesh = plsc.VectorSubcoreMesh(core_axis_name="c", subcore_axis_name="s")
    run = pl.kernel(
        functools.partial(_gather_body, n_ch),
        mesh=mesh,
        out_type=jax.ShapeDtypeStruct((g * _CH, _D), jnp.float32),
        scratch_types=[
            pltpu.VMEM((n_ch, _CH), jnp.int32),
            pltpu.VMEM((_CH, _D), jnp.float32),
            pltpu.SemaphoreType.DMA,
        ],
    )
    return run(table, idx2d)


def _select_body(n_m, inv_s, ksub_ref, qm_ref, idx_ref, key_ref):
    i = pl.program_id(0)
    bh, l = key_ref.shape

    @pl.when(i < n_m)
    def _m_phase():
        rows = []
        for g in range(_GM):
            s = lax.dot_general(ksub_ref[g].astype(jnp.bfloat16),
                                qm_ref[g].astype(jnp.bfloat16),
                                (((1,), (1,)), ((), ())),
                                preferred_element_type=jnp.float32)
            rows.append(jnp.max(s, axis=0, keepdims=True)
                        - jnp.sum(s, axis=0, keepdims=True) * inv_s)
        m = jnp.concatenate(rows, axis=0)
        bits = lax.bitcast_convert_type(m, jnp.int32)
        key = jnp.where(bits < 0, bits ^ jnp.int32(0x7FFFFFFF), bits)
        col = lax.broadcasted_iota(jnp.int32, (_GM, l), 1)
        key_ref[pl.ds(_GM * i, _GM), :] = (
            (key & jnp.int32(~(l - 1))) | (jnp.int32(l - 1) - col))

    @pl.when(i == n_m)
    def _topk_phase():
        col_u = lax.broadcasted_iota(jnp.int32, (bh, _U), 1)

        def body(t, acc):
            kk = key_ref[...]
            mx = jnp.max(kk, axis=1, keepdims=True)
            sel = jnp.int32(l - 1) - (mx & jnp.int32(l - 1))
            key_ref[...] = jnp.where(kk == mx, jnp.int32(_NEG), kk)
            return acc + jnp.where(col_u == t, sel, 0)

        idx_ref[...] = lax.fori_loop(0, _U, body,
                                     jnp.zeros((bh, _U), jnp.int32))


def _select(ksub3, q3):
    bh, l, d = q3.shape
    n_m = bh // _GM
    m_map = lambda i: (jnp.minimum(i, n_m - 1), 0, 0)
    return pl.pallas_call(
        functools.partial(_select_body, n_m, 1.0 / l),
        grid=(n_m + 1,),
        in_specs=[pl.BlockSpec((_GM, _U, d), m_map),
                  pl.BlockSpec((_GM, l, d), m_map)],
        out_specs=pl.BlockSpec((bh, _U), lambda i: (0, 0)),
        out_shape=jax.ShapeDtypeStruct((bh, _U), jnp.int32),
        scratch_shapes=[pltpu.VMEM((bh, l), jnp.int32)],
    )(ksub3, q3)


def _attn_body(scale, qr_ref, k_ref, v_ref, o_ref):
    for g in range(_G):
        v = v_ref[g]
        s = lax.dot_general(qr_ref[g].astype(jnp.bfloat16),
                            k_ref[g].astype(jnp.bfloat16),
                            (((1,), (1,)), ((), ())),
                            preferred_element_type=jnp.float32) * scale
        mx = jnp.max(s, axis=1, keepdims=True)
        e = jnp.exp(s - mx)
        attn = e / jnp.sum(e, axis=1, keepdims=True)
        p1 = lax.dot_general(attn.astype(jnp.bfloat16), v.astype(jnp.bfloat16),
                             (((1,), (0,)), ((), ())),
                             preferred_element_type=jnp.float32)
        vsum = jnp.sum(v, axis=0, keepdims=True)
        fill = jnp.broadcast_to(vsum, (v.shape[0] - _U, v.shape[1]))
        o_ref[g] = jnp.concatenate([p1, fill], axis=0)


def _attention(qr3, k3, v3):
    bh, s, d = k3.shape
    return pl.pallas_call(
        functools.partial(_attn_body, 1.0 / math.sqrt(d)),
        grid=(bh // _G,),
        in_specs=[pl.BlockSpec((_G, _U, d), lambda i: (i, 0, 0)),
                  pl.BlockSpec((_G, s, d), lambda i: (i, 0, 0)),
                  pl.BlockSpec((_G, s, d), lambda i: (i, 0, 0))],
        out_specs=pl.BlockSpec((_G, s, d), lambda i: (i, 0, 0)),
        out_shape=jax.ShapeDtypeStruct((bh, s, d), jnp.float32),
    )(qr3, k3, v3)


def kernel(queries, keys, values):
    b, l, h, d = queries.shape
    s = keys.shape[1]
    bh = b * h
    q3 = jnp.reshape(queries, (bh, l, d))
    k3 = jnp.reshape(keys, (bh, s, d))
    v3 = jnp.reshape(values, (bh, s, d))

    skey = jax.random.key(42)
    _, k2 = jax.random.split(skey)
    idx_k = jax.random.randint(k2, (_U,), 0, s).astype(jnp.int32)

    offs_k = jnp.arange(bh, dtype=jnp.int32)[:, None] * s
    offs_q = jnp.arange(bh, dtype=jnp.int32)[:, None] * l
    ksub = _row_gather(jnp.reshape(k3, (bh * s, d)),
                       offs_k + idx_k[None, :])
    mtop = _select(jnp.reshape(ksub, (bh, _U, d)), q3)
    qr = _row_gather(jnp.reshape(q3, (bh * l, d)), offs_q + mtop)
    out3 = _attention(jnp.reshape(qr, (bh, _U, d)), k3, v3)
    return jnp.reshape(out3, (b, h, s, d))


# speedup vs baseline: 1.0053x; 1.0053x over previous
import functools
import math

import jax
import jax.numpy as jnp
from jax import lax
from jax.experimental import pallas as pl
from jax.experimental.pallas import tpu as pltpu
from jax.experimental.pallas import tpu_sc as plsc

_D = 128
_U = 128
_G = 4
_GM = 8
_NC, _NS = 2, 16
_NW = _NC * _NS
_CH = 128
_NEG = -(2 ** 31)


def _gather_body(n_ch, table_hbm, idx_hbm, out_hbm, idx_v, rows_v, sem):
    wid = lax.axis_index("s") * _NC + lax.axis_index("c")
    pltpu.sync_copy(idx_hbm.at[pl.ds(wid * n_ch, n_ch)], idx_v)
    copies = [pltpu.async_copy(table_hbm.at[idx_v.at[j]], rows_v.at[j], sem)
              for j in range(n_ch)]
    for j in range(n_ch):
        copies[j].wait()
        pltpu.sync_copy(rows_v.at[j],
                        out_hbm.at[pl.ds((wid * n_ch + j) * _CH, _CH)])


def _row_gather(table, idx2d):
    g = idx2d.shape[0]
    n_ch = g // _NW
    mesh = plsc.VectorSubcoreMesh(core_axis_name="c", subcore_axis_name="s")
    run = pl.kernel(
        functools.partial(_gather_body, n_ch),
        mesh=mesh,
        out_type=jax.ShapeDtypeStruct((g * _CH, _D), jnp.float32),
        scratch_types=[
            pltpu.VMEM((n_ch, _CH), jnp.int32),
            pltpu.VMEM((n_ch, _CH, _D), jnp.float32),
            pltpu.SemaphoreType.DMA,
        ],
    )
    return run(table, idx2d)


def _select_body(n_m, inv_s, ksub_ref, qm_ref, idx_ref, key_ref):
    i = pl.program_id(0)
    bh, l = key_ref.shape

    @pl.when(i < n_m)
    def _m_phase():
        rows = []
        for g in range(_GM):
            s = lax.dot_general(ksub_ref[g].astype(jnp.bfloat16),
                                qm_ref[g].astype(jnp.bfloat16),
                                (((1,), (1,)), ((), ())),
                                preferred_element_type=jnp.float32)
            rows.append(jnp.max(s, axis=0, keepdims=True)
                        - jnp.sum(s, axis=0, keepdims=True) * inv_s)
        m = jnp.concatenate(rows, axis=0)
        bits = lax.bitcast_convert_type(m, jnp.int32)
        key = jnp.where(bits < 0, bits ^ jnp.int32(0x7FFFFFFF), bits)
        col = lax.broadcasted_iota(jnp.int32, (_GM, l), 1)
        key_ref[pl.ds(_GM * i, _GM), :] = (
            (key & jnp.int32(~(l - 1))) | (jnp.int32(l - 1) - col))

    @pl.when(i == n_m)
    def _topk_phase():
        col_u = lax.broadcasted_iota(jnp.int32, (bh, _U), 1)

        def body(t, acc):
            kk = key_ref[...]
            mx = jnp.max(kk, axis=1, keepdims=True)
            sel = jnp.int32(l - 1) - (mx & jnp.int32(l - 1))
            key_ref[...] = jnp.where(kk == mx, jnp.int32(_NEG), kk)
            return acc + jnp.where(col_u == t, sel, 0)

        idx_ref[...] = lax.fori_loop(0, _U, body,
                                     jnp.zeros((bh, _U), jnp.int32))


def _select(ksub3, q3):
    bh, l, d = q3.shape
    n_m = bh // _GM
    m_map = lambda i: (jnp.minimum(i, n_m - 1), 0, 0)
    return pl.pallas_call(
        functools.partial(_select_body, n_m, 1.0 / l),
        grid=(n_m + 1,),
        in_specs=[pl.BlockSpec((_GM, _U, d), m_map),
                  pl.BlockSpec((_GM, l, d), m_map)],
        out_specs=pl.BlockSpec((bh, _U), lambda i: (0, 0)),
        out_shape=jax.ShapeDtypeStruct((bh, _U), jnp.int32),
        scratch_shapes=[pltpu.VMEM((bh, l), jnp.int32)],
    )(ksub3, q3)


def _attn_body(scale, qr_ref, k_ref, v_ref, o_ref):
    for g in range(_G):
        v = v_ref[g]
        s = lax.dot_general(qr_ref[g].astype(jnp.bfloat16),
                            k_ref[g].astype(jnp.bfloat16),
                            (((1,), (1,)), ((), ())),
                            preferred_element_type=jnp.float32) * scale
        mx = jnp.max(s, axis=1, keepdims=True)
        e = jnp.exp(s - mx)
        attn = e / jnp.sum(e, axis=1, keepdims=True)
        p1 = lax.dot_general(attn.astype(jnp.bfloat16), v.astype(jnp.bfloat16),
                             (((1,), (0,)), ((), ())),
                             preferred_element_type=jnp.float32)
        vsum = jnp.sum(v, axis=0, keepdims=True)
        fill = jnp.broadcast_to(vsum, (v.shape[0] - _U, v.shape[1]))
        o_ref[g] = jnp.concatenate([p1, fill], axis=0)


def _attention(qr3, k3, v3):
    bh, s, d = k3.shape
    return pl.pallas_call(
        functools.partial(_attn_body, 1.0 / math.sqrt(d)),
        grid=(bh // _G,),
        in_specs=[pl.BlockSpec((_G, _U, d), lambda i: (i, 0, 0)),
                  pl.BlockSpec((_G, s, d), lambda i: (i, 0, 0)),
                  pl.BlockSpec((_G, s, d), lambda i: (i, 0, 0))],
        out_specs=pl.BlockSpec((_G, s, d), lambda i: (i, 0, 0)),
        out_shape=jax.ShapeDtypeStruct((bh, s, d), jnp.float32),
    )(qr3, k3, v3)


def kernel(queries, keys, values):
    b, l, h, d = queries.shape
    s = keys.shape[1]
    bh = b * h
    q3 = jnp.reshape(queries, (bh, l, d))
    k3 = jnp.reshape(keys, (bh, s, d))
    v3 = jnp.reshape(values, (bh, s, d))

    skey = jax.random.key(42)
    _, k2 = jax.random.split(skey)
    idx_k = jax.random.randint(k2, (_U,), 0, s).astype(jnp.int32)

    offs_k = jnp.arange(bh, dtype=jnp.int32)[:, None] * s
    offs_q = jnp.arange(bh, dtype=jnp.int32)[:, None] * l
    ksub = _row_gather(jnp.reshape(k3, (bh * s, d)),
                       offs_k + idx_k[None, :])
    mtop = _select(jnp.reshape(ksub, (bh, _U, d)), q3)
    qr = _row_gather(jnp.reshape(q3, (bh * l, d)), offs_q + mtop)
    out3 = _attention(jnp.reshape(qr, (bh, _U, d)), k3, v3)
    return jnp.reshape(out3, (b, h, s, d))


# speedup vs baseline: 1.0157x; 1.0103x over previous
import functools
import math

import jax
import jax.numpy as jnp
from jax import lax
from jax.experimental import pallas as pl
from jax.experimental.pallas import tpu as pltpu
from jax.experimental.pallas import tpu_sc as plsc

_D = 128
_U = 128
_G = 4
_GM = 8
_NC, _NS = 2, 16
_NW = _NC * _NS
_CH = 128
_NEG = -(2 ** 31)


def _gather_body(n_ch, table_hbm, idx_hbm, out_hbm, idx_v, rows_v, sem):
    wid = lax.axis_index("s") * _NC + lax.axis_index("c")
    pltpu.sync_copy(idx_hbm.at[pl.ds(wid * n_ch, n_ch)], idx_v)
    copies = [pltpu.async_copy(table_hbm.at[idx_v.at[j]], rows_v.at[j], sem)
              for j in range(n_ch)]
    for j in range(n_ch):
        copies[j].wait()
        pltpu.sync_copy(rows_v.at[j],
                        out_hbm.at[pl.ds((wid * n_ch + j) * _CH, _CH)])


def _row_gather(table, idx2d):
    g = idx2d.shape[0]
    n_ch = g // _NW
    mesh = plsc.VectorSubcoreMesh(core_axis_name="c", subcore_axis_name="s")
    run = pl.kernel(
        functools.partial(_gather_body, n_ch),
        mesh=mesh,
        out_type=jax.ShapeDtypeStruct((g * _CH, _D), jnp.float32),
        scratch_types=[
            pltpu.VMEM((n_ch, _CH), jnp.int32),
            pltpu.VMEM((n_ch, _CH, _D), jnp.float32),
            pltpu.SemaphoreType.DMA,
        ],
    )
    return run(table, idx2d)


def _select_body(n_m, inv_s, ksub_ref, qm_ref, idx_ref, key_ref):
    i = pl.program_id(0)
    bh, l = key_ref.shape

    @pl.when(i < n_m)
    def _m_phase():
        rows = []
        for g in range(_GM):
            s = lax.dot_general(ksub_ref[g].astype(jnp.bfloat16),
                                qm_ref[g].astype(jnp.bfloat16),
                                (((1,), (1,)), ((), ())),
                                preferred_element_type=jnp.float32)
            rows.append(jnp.max(s, axis=0, keepdims=True)
                        - jnp.sum(s, axis=0, keepdims=True) * inv_s)
        m = jnp.concatenate(rows, axis=0)
        bits = lax.bitcast_convert_type(m, jnp.int32)
        key = jnp.where(bits < 0, bits ^ jnp.int32(0x7FFFFFFF), bits)
        col = lax.broadcasted_iota(jnp.int32, (_GM, l), 1)
        key_ref[pl.ds(_GM * i, _GM), :] = (
            (key & jnp.int32(~(l - 1))) | (jnp.int32(l - 1) - col))

    @pl.when(i == n_m)
    def _topk_phase():
        col_u = lax.broadcasted_iota(jnp.int32, (bh, _U), 1)

        def body(t, acc):
            kk = key_ref[...]
            mx = jnp.max(kk, axis=1, keepdims=True)
            sel = jnp.int32(l - 1) - (mx & jnp.int32(l - 1))
            key_ref[...] = jnp.where(kk == mx, jnp.int32(_NEG), kk)
            return acc + jnp.where(col_u == t, sel, 0)

        idx_ref[...] = (lax.fori_loop(0, _U, body,
                                      jnp.zeros((bh, _U), jnp.int32))
                        + lax.broadcasted_iota(jnp.int32, (bh, _U), 0)
                        * jnp.int32(l))


def _select(ksub3, q3):
    bh, l, d = q3.shape
    n_m = bh // _GM
    m_map = lambda i: (jnp.minimum(i, n_m - 1), 0, 0)
    return pl.pallas_call(
        functools.partial(_select_body, n_m, 1.0 / l),
        grid=(n_m + 1,),
        in_specs=[pl.BlockSpec((_GM, _U, d), m_map),
                  pl.BlockSpec((_GM, l, d), m_map)],
        out_specs=pl.BlockSpec((bh, _U), lambda i: (0, 0)),
        out_shape=jax.ShapeDtypeStruct((bh, _U), jnp.int32),
        scratch_shapes=[pltpu.VMEM((bh, l), jnp.int32)],
    )(ksub3, q3)


def _attn_body(scale, qr_ref, k_ref, v_ref, o_ref):
    for g in range(_G):
        v = v_ref[g]
        s = lax.dot_general(qr_ref[g].astype(jnp.bfloat16),
                            k_ref[g].astype(jnp.bfloat16),
                            (((1,), (1,)), ((), ())),
                            preferred_element_type=jnp.float32) * scale
        mx = jnp.max(s, axis=1, keepdims=True)
        e = jnp.exp(s - mx)
        attn = e / jnp.sum(e, axis=1, keepdims=True)
        p1 = lax.dot_general(attn.astype(jnp.bfloat16), v.astype(jnp.bfloat16),
                             (((1,), (0,)), ((), ())),
                             preferred_element_type=jnp.float32)
        vsum = jnp.sum(v, axis=0, keepdims=True)
        fill = jnp.broadcast_to(vsum, (v.shape[0] - _U, v.shape[1]))
        o_ref[g] = jnp.concatenate([p1, fill], axis=0)


def _attention(qr3, k3, v3):
    bh, s, d = k3.shape
    return pl.pallas_call(
        functools.partial(_attn_body, 1.0 / math.sqrt(d)),
        grid=(bh // _G,),
        in_specs=[pl.BlockSpec((_G, _U, d), lambda i: (i, 0, 0)),
                  pl.BlockSpec((_G, s, d), lambda i: (i, 0, 0)),
                  pl.BlockSpec((_G, s, d), lambda i: (i, 0, 0))],
        out_specs=pl.BlockSpec((_G, s, d), lambda i: (i, 0, 0)),
        out_shape=jax.ShapeDtypeStruct((bh, s, d), jnp.float32),
    )(qr3, k3, v3)


def kernel(queries, keys, values):
    b, l, h, d = queries.shape
    s = keys.shape[1]
    bh = b * h
    q3 = jnp.reshape(queries, (bh, l, d))
    k3 = jnp.reshape(keys, (bh, s, d))
    v3 = jnp.reshape(values, (bh, s, d))

    skey = jax.random.key(42)
    _, k2 = jax.random.split(skey)
    idx_k = jax.random.randint(k2, (_U,), 0, s).astype(jnp.int32)

    offs_k = jnp.arange(bh, dtype=jnp.int32)[:, None] * s
    ksub = _row_gather(jnp.reshape(k3, (bh * s, d)),
                       offs_k + idx_k[None, :])
    mtop = _select(jnp.reshape(ksub, (bh, _U, d)), q3)
    qr = _row_gather(jnp.reshape(q3, (bh * l, d)), mtop)
    out3 = _attention(jnp.reshape(qr, (bh, _U, d)), k3, v3)
    return jnp.reshape(out3, (b, h, s, d))
